# Initial kernel scaffold; baseline (speedup 1.0000x reference)
import jax
import jax.numpy as jnp
from jax.experimental import pallas as pl


def kernel(x, table):
    raise NotImplementedError("write your pallas kernel here")


# speedup vs baseline: 1.1088x; 1.1088x over previous
import functools

import jax
import jax.numpy as jnp
from jax import lax
from jax.experimental import pallas as pl
from jax.experimental.pallas import tpu as pltpu
from jax.experimental.pallas import tpu_sc as plsc

DIM = 32
C = 128
K = 8
G = K * C
NC = 2
NS = 16
NW = NC * NS
NG = 25
NI = 12
B_TOTAL = NW * NG * G


def _emb_kernel(table_hbm, idx_hbm, out_hbm, idx_v, rows_v, gsem0, gsem1):
    wid = lax.axis_index("s") * NC + lax.axis_index("c")
    gbase = wid * NG

    def fire(g, b, gsem):
        pltpu.sync_copy(idx_hbm.at[pl.ds(g * K, K)], idx_v.at[b])
        return [
            pltpu.async_copy(
                table_hbm.at[idx_v.at[b, j]],
                rows_v.at[b, pl.ds(j * C, C)],
                gsem,
            )
            for j in range(K)
        ]

    def drain(g, b, descs):
        for d in descs:
            d.wait()
        pltpu.sync_copy(rows_v.at[b], out_hbm.at[pl.ds(g * G, G)])

    def iter_body(i, carry):
        g0 = gbase + 2 * i
        d0 = fire(g0, 0, gsem0)
        d1 = fire(g0 + 1, 1, gsem1)
        drain(g0, 0, d0)
        drain(g0 + 1, 1, d1)
        return carry

    lax.fori_loop(0, NI, iter_body, 0)
    g_last = gbase + 2 * NI
    drain(g_last, 0, fire(g_last, 0, gsem0))


@jax.jit
def kernel(x, table):
    batch, hist = x.shape
    assert batch * hist == B_TOTAL and table.shape[1] == DIM
    idx = x.reshape(B_TOTAL // C, C).astype(jnp.int32)

    run = functools.partial(
        pl.kernel,
        out_type=jax.ShapeDtypeStruct((B_TOTAL, DIM), jnp.float32),
        mesh=plsc.VectorSubcoreMesh(core_axis_name="c", subcore_axis_name="s"),
        compiler_params=pltpu.CompilerParams(use_tc_tiling_on_sc=False),
        scratch_types=[
            pltpu.VMEM((2, K, C), jnp.int32),
            pltpu.VMEM((2, G, DIM), jnp.float32),
            pltpu.SemaphoreType.DMA,
            pltpu.SemaphoreType.DMA,
        ],
    )(_emb_kernel)

    out = run(table, idx)
    return out.reshape(batch, hist, DIM)


# speedup vs baseline: 1.7460x; 1.5746x over previous
import functools

import jax
import jax.numpy as jnp
from jax import lax
from jax.experimental import pallas as pl
from jax.experimental.pallas import tpu as pltpu
from jax.experimental.pallas import tpu_sc as plsc

BATCH = 16384
HIST = 50
DIM = 32
NC = 2
NS = 16
NW = NC * NS
R = 8
NG = BATCH // (NW * R)
NI = NG // 2


def _emb_kernel(table_hbm, x_hbm, out_hbm, idx_v, rows_v, gsem0, gsem1):
    wid = lax.axis_index("s") * NC + lax.axis_index("c")
    rbase = wid * NG * R

    def fire(r0, b, gsem):
        pltpu.sync_copy(x_hbm.at[pl.ds(r0, R)], idx_v.at[b])
        return [
            pltpu.async_copy(
                table_hbm.at[idx_v.at[b, r]], rows_v.at[b, r], gsem
            )
            for r in range(R)
        ]

    def drain(r0, b, descs):
        for d in descs:
            d.wait()
        pltpu.sync_copy(rows_v.at[b], out_hbm.at[pl.ds(r0, R)])

    def iter_body(i, carry):
        r0 = rbase + 2 * i * R
        d0 = fire(r0, 0, gsem0)
        d1 = fire(r0 + R, 1, gsem1)
        drain(r0, 0, d0)
        drain(r0 + R, 1, d1)
        return carry

    lax.fori_loop(0, NI, iter_body, 0)


@jax.jit
def kernel(x, table):
    assert x.shape == (BATCH, HIST) and table.shape[1] == DIM

    run = functools.partial(
        pl.kernel,
        out_type=jax.ShapeDtypeStruct((BATCH, HIST, DIM), jnp.float32),
        mesh=plsc.VectorSubcoreMesh(core_axis_name="c", subcore_axis_name="s"),
        compiler_params=pltpu.CompilerParams(use_tc_tiling_on_sc=False),
        scratch_types=[
            pltpu.VMEM((2, R, HIST), jnp.int32),
            pltpu.VMEM((2, R, HIST, DIM), jnp.float32),
            pltpu.SemaphoreType.DMA,
            pltpu.SemaphoreType.DMA,
        ],
    )(_emb_kernel)

    return run(table, x.astype(jnp.int32))


# speedup vs baseline: 1.7474x; 1.0008x over previous
import functools

import jax
import jax.numpy as jnp
from jax import lax
from jax.experimental import layout as jlayout
from jax.experimental import pallas as pl
from jax.experimental.pallas import tpu as pltpu
from jax.experimental.pallas import tpu_sc as plsc

BATCH = 16384
HIST = 50
DIM = 32
NC = 2
NS = 16
NW = NC * NS
R = 8
NG = BATCH // (NW * R)
NI = NG // 2


def _emb_kernel(table_hbm, x_hbm, out_hbm, idx_v, rows_v, gsem0, gsem1):
    wid = lax.axis_index("s") * NC + lax.axis_index("c")
    rbase = wid * NG * R

    def fire(r0, b, gsem):
        pltpu.sync_copy(x_hbm.at[pl.ds(r0, R)], idx_v.at[b])
        return [
            pltpu.async_copy(
                table_hbm.at[idx_v.at[b, r]], rows_v.at[b, r], gsem
            )
            for r in range(R)
        ]

    def drain(r0, b, descs):
        for d in descs:
            d.wait()
        pltpu.sync_copy(rows_v.at[b], out_hbm.at[pl.ds(r0, R)])

    def iter_body(i, carry):
        r0 = rbase + 2 * i * R
        d0 = fire(r0, 0, gsem0)
        d1 = fire(r0 + R, 1, gsem1)
        drain(r0, 0, d0)
        drain(r0 + R, 1, d1)
        return carry

    lax.fori_loop(0, NI, iter_body, 0)


@jax.jit
def kernel(x, table):
    assert x.shape == (BATCH, HIST) and table.shape[1] == DIM

    run = functools.partial(
        pl.kernel,
        out_type=jax.ShapeDtypeStruct((BATCH, HIST, DIM), jnp.float32),
        mesh=plsc.VectorSubcoreMesh(core_axis_name="c", subcore_axis_name="s"),
        compiler_params=pltpu.CompilerParams(use_tc_tiling_on_sc=False),
        scratch_types=[
            pltpu.VMEM((2, R, HIST), jnp.int32),
            pltpu.VMEM((2, R, HIST, DIM), jnp.float32),
            pltpu.SemaphoreType.DMA,
            pltpu.SemaphoreType.DMA,
        ],
    )(_emb_kernel)

    out = run(table, x.astype(jnp.int32))
    return jlayout.with_layout_constraint(
        out, jlayout.Layout(major_to_minor=(0, 1, 2), tiling=())
    )
